# Initial kernel scaffold; baseline (speedup 1.0000x reference)
import functools

import jax
import jax.numpy as jnp
from jax import lax
from jax.experimental import pallas as pl
from jax.experimental.pallas import tpu as pltpu
from jax.experimental.pallas import tpu_sc as plsc

EMB_DIM = 64
CHUNK = 128
GROUP = 512
NW = 32


@functools.lru_cache(maxsize=None)
def _build(B: int):
    b_per_w = B // NW
    n_groups = b_per_w // GROUP
    k = GROUP // CHUNK
    mesh = plsc.VectorSubcoreMesh(core_axis_name="c", subcore_axis_name="s")

    @functools.partial(
        pl.kernel,
        mesh=mesh,
        out_type=jax.ShapeDtypeStruct((B, EMB_DIM), jnp.float32),
        scratch_types=[
            pltpu.VMEM((b_per_w // CHUNK, CHUNK), jnp.int32),
            pltpu.VMEM((GROUP, EMB_DIM), jnp.float32),
            pltpu.SemaphoreType.DMA,
        ],
    )
    def kern(idx_hbm, table_hbm, out_hbm, idx_v, rows_v, sem):
        wid = lax.axis_index("s") * 2 + lax.axis_index("c")
        row0 = wid * (b_per_w // CHUNK)
        base = wid * b_per_w
        pltpu.sync_copy(idx_hbm.at[pl.ds(row0, b_per_w // CHUNK)], idx_v)

        def body(g, carry):
            cps = [
                pltpu.async_copy(
                    table_hbm.at[idx_v.at[g * k + c]],
                    rows_v.at[pl.ds(c * CHUNK, CHUNK)],
                    sem,
                )
                for c in range(k)
            ]
            for cp in cps:
                cp.wait()
            pltpu.sync_copy(rows_v, out_hbm.at[pl.ds(base + g * GROUP, GROUP)])
            return carry

        lax.fori_loop(0, n_groups, body, 0)

    return kern


def kernel(inputs, weight):
    batch, hist = inputs.shape
    B = batch * hist
    idx = inputs.reshape(B // CHUNK, CHUNK).astype(jnp.int32)
    out = _build(B)(idx, weight)
    return out.reshape(batch, hist, EMB_DIM)


# speedup vs baseline: 1.8326x; 1.8326x over previous
import functools

import jax
import jax.numpy as jnp
from jax import lax
from jax.experimental import pallas as pl
from jax.experimental.pallas import tpu as pltpu
from jax.experimental.pallas import tpu_sc as plsc

EMB_DIM = 64
CHUNK = 128
GROUP = 512
NW = 32


@functools.lru_cache(maxsize=None)
def _build(B: int):
    b_per_w = B // NW
    n_groups = b_per_w // GROUP
    k = GROUP // CHUNK
    mesh = plsc.VectorSubcoreMesh(core_axis_name="c", subcore_axis_name="s")

    @functools.partial(
        pl.kernel,
        mesh=mesh,
        compiler_params=pltpu.CompilerParams(use_tc_tiling_on_sc=False),
        out_type=jax.ShapeDtypeStruct((B, EMB_DIM), jnp.float32),
        scratch_types=[
            pltpu.VMEM((b_per_w // CHUNK, CHUNK), jnp.int32),
            pltpu.VMEM((GROUP, EMB_DIM), jnp.float32),
            pltpu.SemaphoreType.DMA,
        ],
    )
    def kern(idx_hbm, table_hbm, out_hbm, idx_v, rows_v, sem):
        wid = lax.axis_index("s") * 2 + lax.axis_index("c")
        row0 = wid * (b_per_w // CHUNK)
        base = wid * b_per_w
        pltpu.sync_copy(idx_hbm.at[pl.ds(row0, b_per_w // CHUNK)], idx_v)

        def body(g, carry):
            cps = [
                pltpu.async_copy(
                    table_hbm.at[idx_v.at[g * k + c]],
                    rows_v.at[pl.ds(c * CHUNK, CHUNK)],
                    sem,
                )
                for c in range(k)
            ]
            for cp in cps:
                cp.wait()
            pltpu.sync_copy(rows_v, out_hbm.at[pl.ds(base + g * GROUP, GROUP)])
            return carry

        lax.fori_loop(0, n_groups, body, 0)

    return kern


def kernel(inputs, weight):
    batch, hist = inputs.shape
    B = batch * hist
    idx = inputs.reshape(B // CHUNK, CHUNK).astype(jnp.int32)
    out = _build(B)(idx, weight)
    return out.reshape(batch, hist, EMB_DIM)


# speedup vs baseline: 1.8764x; 1.0239x over previous
import functools

import jax
import jax.numpy as jnp
from jax import lax
from jax.experimental import pallas as pl
from jax.experimental.pallas import tpu as pltpu
from jax.experimental.pallas import tpu_sc as plsc

EMB_DIM = 64
CHUNK = 128
GROUP = 256
NBUF = 4
NW = 32
K = GROUP // CHUNK


@functools.lru_cache(maxsize=None)
def _build(B: int):
    b_per_w = B // NW
    n_groups = b_per_w // GROUP
    assert n_groups % NBUF == 0 and n_groups > 2 * NBUF
    mesh = plsc.VectorSubcoreMesh(core_axis_name="c", subcore_axis_name="s")

    @functools.partial(
        pl.kernel,
        mesh=mesh,
        compiler_params=pltpu.CompilerParams(use_tc_tiling_on_sc=False),
        out_type=jax.ShapeDtypeStruct((B, EMB_DIM), jnp.float32),
        scratch_types=[
            pltpu.VMEM((b_per_w // CHUNK, CHUNK), jnp.int32),
            *[pltpu.VMEM((GROUP, EMB_DIM), jnp.float32) for _ in range(NBUF)],
            *[pltpu.SemaphoreType.DMA for _ in range(2 * NBUF)],
        ],
    )
    def kern(idx_hbm, table_hbm, out_hbm, idx_v, *bufs_and_sems):
        rows = bufs_and_sems[:NBUF]
        sem_g = bufs_and_sems[NBUF:2 * NBUF]
        sem_s = bufs_and_sems[2 * NBUF:]
        wid = lax.axis_index("s") * 2 + lax.axis_index("c")
        base = wid * b_per_w
        pltpu.sync_copy(idx_hbm.at[pl.ds(wid * (b_per_w // CHUNK),
                                         b_per_w // CHUNK)], idx_v)

        def start_gather(g, b):
            for c in range(K):
                pltpu.async_copy(
                    table_hbm.at[idx_v.at[g * K + c]],
                    rows[b].at[pl.ds(c * CHUNK, CHUNK)],
                    sem_g[b],
                )

        def wait_gather(b):
            for c in range(K):
                pltpu.make_async_copy(
                    table_hbm.at[idx_v.at[c]],
                    rows[b].at[pl.ds(c * CHUNK, CHUNK)],
                    sem_g[b],
                ).wait()

        def start_store(g, b):
            pltpu.async_copy(rows[b], out_hbm.at[pl.ds(base + g * GROUP, GROUP)],
                             sem_s[b])

        def wait_store(b):
            pltpu.make_async_copy(rows[b], out_hbm.at[pl.ds(base, GROUP)],
                                  sem_s[b]).wait()

        start_gather(0, 0)
        for g in range(NBUF - 1):
            start_gather(g + 1, (g + 1) % NBUF)
            wait_gather(g % NBUF)
            start_store(g, g % NBUF)

        def body(i, carry):
            for j in range(NBUF):
                g = (NBUF - 1) + i * NBUF + j
                b = (NBUF - 1 + j) % NBUF
                bn = (b + 1) % NBUF
                wait_store(bn)
                start_gather(g + 1, bn)
                wait_gather(b)
                start_store(g, b)
            return carry

        lax.fori_loop(0, (n_groups - NBUF) // NBUF, body, 0)

        g_last = n_groups - 1
        wait_gather(g_last % NBUF)
        start_store(g_last, g_last % NBUF)
        for b in range(NBUF):
            wait_store(b)

    return kern


def kernel(inputs, weight):
    batch, hist = inputs.shape
    B = batch * hist
    idx = inputs.reshape(B // CHUNK, CHUNK).astype(jnp.int32)
    out = _build(B)(idx, weight)
    return out.reshape(batch, hist, EMB_DIM)
